# baseline (speedup 1.0000x reference)
import functools

import jax
import jax.numpy as jnp
from jax import lax
from jax.experimental import pallas as pl
from jax.experimental.pallas import tpu as pltpu
from jax.experimental.pallas import tpu_sc as plsc

K = 1024
D = 32
N_TOK = 32 * 32 * 32 * 32 // D

T_BLK = 2048
N_BLK = N_TOK // T_BLK

NC, NS = 2, 16
NW = NC * NS
BPW = N_TOK // NW
L = 16


def _argmin_body(x_ref, e_ref, idx_ref):
    x = x_ref[...]
    e = e_ref[...]
    scores = lax.dot_general(x, e, (((1,), (0,)), ((), ())),
                             preferred_element_type=jnp.float32)
    en2 = jnp.sum(e * e, axis=0, keepdims=True)
    dist = en2 - 2.0 * scores
    idx = jnp.argmin(dist, axis=1).astype(jnp.int32)
    idx_ref[...] = idx.reshape(1, 1, T_BLK)


def _tc_argmin(x2d, embeddings):
    return pl.pallas_call(
        _argmin_body,
        grid=(N_BLK,),
        in_specs=[
            pl.BlockSpec((T_BLK, D), lambda i: (i, 0)),
            pl.BlockSpec((D, K), lambda i: (0, 0)),
        ],
        out_specs=pl.BlockSpec((1, 1, T_BLK), lambda i: (i, 0, 0)),
        out_shape=jax.ShapeDtypeStruct((N_BLK, 1, T_BLK), jnp.int32),
    )(x2d, embeddings)


@functools.partial(
    pl.kernel,
    mesh=plsc.VectorSubcoreMesh(core_axis_name="c", subcore_axis_name="s"),
    out_type=jax.ShapeDtypeStruct((NW, BPW * D), jnp.float32),
    scratch_types=[
        pltpu.VMEM((K * D,), jnp.float32),
        pltpu.VMEM((BPW,), jnp.int32),
        pltpu.VMEM((BPW * D,), jnp.float32),
    ],
    compiler_params=pltpu.CompilerParams(needs_layout_passes=False),
)
def _sc_gather(table_hbm, idx_hbm, out_hbm, table_v, idx_v, out_v):
    wid = lax.axis_index("s") * NC + lax.axis_index("c")
    pltpu.sync_copy(table_hbm, table_v)
    pltpu.sync_copy(idx_hbm.at[wid], idx_v)
    lane = lax.iota(jnp.int32, L)
    lane_off = lane * D
    def body(g, carry):
        tok = idx_v[pl.ds(g * L, L)]
        gbase = tok * D
        sbase = lane_off + g * (L * D)
        for d in range(D):
            vals = plsc.load_gather(table_v, [gbase + d])
            plsc.store_scatter(out_v, [sbase + d], vals)
        return carry
    lax.fori_loop(0, BPW // L, body, 0)
    pltpu.sync_copy(out_v, out_hbm.at[wid])


def kernel(x, embeddings):
    x2d = x.reshape(N_TOK, D)
    idx = _tc_argmin(x2d, embeddings).reshape(NW, BPW)
    table = embeddings.T.reshape(K * D)
    q = _sc_gather(table, idx)
    return q.reshape(x.shape)
